# Initial kernel scaffold; baseline (speedup 1.0000x reference)
import jax
import jax.numpy as jnp
from jax.experimental import pallas as pl


def kernel(x, edge_index, c, W1, b1, g1, be1, W2, b2, eps1, W3, b3, g2, be2, W4, b4, eps2):
    raise NotImplementedError("write your pallas kernel here")


# speedup vs baseline: 7.2424x; 7.2424x over previous
import functools

import jax
import jax.numpy as jnp
from jax import lax
from jax.experimental import pallas as pl
from jax.experimental.pallas import tpu as pltpu
from jax.experimental.pallas import tpu_sc as plsc

_N = 50000
_E = 800000

_NTILES = 16
_CHUNK = 512
_JROWS = 4
_EPT = 51200
_EPAD = _EPT * _NTILES
_NITER = _EPT // _CHUNK
_ACC_ROWS = 50176
_ZROWS = 3136
_COPY_N = 50048
_CROWS = _COPY_N // _NTILES
_GARBAGE = 50100


def _seg_body(x_hbm, c2_hbm, src2d, dst2d, sx_out, sc_out,
              idx_s, idx_d, rows, zbuf, acc, sem_g, sem_s):
    cid = lax.axis_index("c")
    sid = lax.axis_index("s")

    zv = jnp.zeros((16,), jnp.float32)

    def zero_row(i, _):
        zbuf[i, pl.ds(0, 16)] = zv
        zbuf[i, pl.ds(16, 16)] = zv
        return 0

    lax.fori_loop(0, 64, zero_row, 0)
    zbase = sid * _ZROWS

    def zero_acc(k, _):
        pltpu.sync_copy(zbuf, acc.at[pl.ds(zbase + 64 * k, 64)])
        return 0

    lax.fori_loop(0, _ZROWS // 64, zero_acc, 0)
    plsc.subcore_barrier()

    def step(i, _):
        row0 = sid * (_EPT // 128) + i * _JROWS
        pltpu.sync_copy(src2d.at[pl.ds(row0, _JROWS)], idx_s)
        pltpu.sync_copy(dst2d.at[pl.ds(row0, _JROWS)], idx_d)

        @pl.when(cid == 0)
        def _():
            cps = [pltpu.async_copy(x_hbm.at[idx_s.at[j]], rows.at[j], sem_g)
                   for j in range(_JROWS)]
            for cp in cps:
                cp.wait()

        @pl.when(cid == 1)
        def _():
            cps = [pltpu.async_copy(c2_hbm.at[idx_s.at[j]], rows.at[j], sem_g)
                   for j in range(_JROWS)]
            for cp in cps:
                cp.wait()

        cps = [pltpu.async_copy(rows.at[j], acc.at[idx_d.at[j]], sem_s,
                                add=True)
               for j in range(_JROWS)]
        for cp in cps:
            cp.wait()
        return 0

    lax.fori_loop(0, _NITER, step, 0)
    plsc.subcore_barrier()

    r0 = sid * _CROWS

    @pl.when(cid == 0)
    def _():
        pltpu.sync_copy(acc.at[pl.ds(r0, _CROWS)], sx_out.at[pl.ds(r0, _CROWS)])

    @pl.when(cid == 1)
    def _():
        pltpu.sync_copy(acc.at[pl.ds(r0, _CROWS)], sc_out.at[pl.ds(r0, _CROWS)])


_seg_call = pl.kernel(
    _seg_body,
    out_type=(jax.ShapeDtypeStruct((_COPY_N, 32), jnp.float32),
              jax.ShapeDtypeStruct((_COPY_N, 32), jnp.float32)),
    mesh=plsc.VectorSubcoreMesh(core_axis_name="c", subcore_axis_name="s"),
    scratch_types=[
        pltpu.VMEM((_JROWS, 128), jnp.int32),
        pltpu.VMEM((_JROWS, 128), jnp.int32),
        pltpu.VMEM((_JROWS, 128, 32), jnp.float32),
        pltpu.VMEM((64, 32), jnp.float32),
        pltpu.VMEM_SHARED((_ACC_ROWS, 32), jnp.float32),
        pltpu.SemaphoreType.DMA,
        pltpu.SemaphoreType.DMA,
    ],
    compiler_params=pltpu.CompilerParams(use_tc_tiling_on_sc=False),
)


_R = 1000
_G = _N // _R


def _p1_body(x_b, c2_b, sx_b, sc_b, a_b, W1x_b, W1c_b, W3x_b, W3c_b,
             b1_b, b3_b, zz0_b, zz1_b, h2a_b, sums_b):
    a1 = a_b[0, 0]
    a2 = a_b[0, 1]
    x = x_b[...]
    sx = sx_b[...]
    c0 = c2_b[:, :16]
    c1 = c2_b[:, 16:]
    s0 = sc_b[:, :16]
    s1 = sc_b[:, 16:]
    xs1 = (a1 * x + sx) @ W1x_b[...]
    zz0 = xs1 + (a1 * c0 + s0) @ W1c_b[...] + b1_b[...]
    zz1 = xs1 + (a1 * c1 + s1) @ W1c_b[...] + b1_b[...]
    h2a = ((a2 * x + sx) @ W3x_b[...]
           + (a2 * (c0 + c1) + (s0 + s1)) @ (0.5 * W3c_b[...])
           + b3_b[...])
    zz0_b[...] = zz0
    zz1_b[...] = zz1
    h2a_b[...] = h2a
    r0 = jnp.sum(zz0, axis=0) + jnp.sum(zz1, axis=0)
    r1 = jnp.sum(zz0 * zz0, axis=0) + jnp.sum(zz1 * zz1, axis=0)
    r2 = jnp.sum(h2a, axis=0)
    r3 = jnp.sum(h2a * h2a, axis=0)
    z = jnp.zeros_like(r0)
    part = jnp.stack([r0, r1, r2, r3, z, z, z, z])

    @pl.when(pl.program_id(0) == 0)
    def _():
        sums_b[...] = jnp.zeros_like(sums_b)

    sums_b[...] += part


def _p2_body(zz0_b, zz1_b, h2a_b, sums_b, W2_b, W4_b, b2_b, b4_b,
             g1_b, be1_b, g2_b, be2_b, out_b):
    sums = sums_b[...]
    m1 = sums[0] * (1.0 / (2 * _N))
    v1 = sums[1] * (1.0 / (2 * _N)) - m1 * m1
    k1 = g1_b[0] * lax.rsqrt(v1 + 1e-5)
    m2 = sums[2] * (1.0 / _N)
    v2 = sums[3] * (1.0 / _N) - m2 * m2
    k2 = g2_b[0] * lax.rsqrt(v2 + 1e-5)
    n0 = jnp.maximum((zz0_b[...] - m1) * k1 + be1_b[...], 0.0) @ W2_b[...] + b2_b[...]
    n1 = jnp.maximum((zz1_b[...] - m1) * k1 + be1_b[...], 0.0) @ W2_b[...] + b2_b[...]
    h2 = jnp.maximum((h2a_b[...] - m2) * k2 + be2_b[...], 0.0) @ W4_b[...] + b4_b[...]
    out_b[...] = jnp.concatenate([n0 + h2, n1 + h2], axis=1)


def _row_blk(w):
    return pl.BlockSpec((_R, w), lambda i: (i, 0))


def _full(shape):
    return pl.BlockSpec(shape, lambda i: tuple(0 for _ in shape))


_p1_call = pl.pallas_call(
    _p1_body,
    grid=(_G,),
    in_specs=[
        _row_blk(32), _row_blk(32), _row_blk(32), _row_blk(32),
        _full((1, 2)),
        _full((32, 64)), _full((16, 64)), _full((32, 64)), _full((16, 64)),
        _full((1, 64)), _full((1, 64)),
    ],
    out_specs=[_row_blk(64), _row_blk(64), _row_blk(64), _full((8, 64))],
    out_shape=[
        jax.ShapeDtypeStruct((_N, 64), jnp.float32),
        jax.ShapeDtypeStruct((_N, 64), jnp.float32),
        jax.ShapeDtypeStruct((_N, 64), jnp.float32),
        jax.ShapeDtypeStruct((8, 64), jnp.float32),
    ],
)

_p2_call = pl.pallas_call(
    _p2_body,
    grid=(_G,),
    in_specs=[
        _row_blk(64), _row_blk(64), _row_blk(64),
        _full((8, 64)),
        _full((64, 64)), _full((64, 64)),
        _full((1, 64)), _full((1, 64)),
        _full((1, 64)), _full((1, 64)), _full((1, 64)), _full((1, 64)),
    ],
    out_specs=_row_blk(128),
    out_shape=jax.ShapeDtypeStruct((_N, 128), jnp.float32),
)


def kernel(x, edge_index, c, W1, b1, g1, be1, W2, b2, eps1, W3, b3, g2, be2, W4, b4, eps2):
    src = edge_index[0]
    dst = edge_index[1]
    pad = _EPAD - _E
    src_p = jnp.concatenate([src, jnp.zeros((pad,), jnp.int32)])
    dst_p = jnp.concatenate([dst, jnp.full((pad,), _GARBAGE, jnp.int32)])
    src2d = src_p.reshape(_EPAD // 128, 128)
    dst2d = dst_p.reshape(_EPAD // 128, 128)
    c2 = c.reshape(_N, 32)

    sx, sc = _seg_call(x, c2, src2d, dst2d)
    sx = sx[:_N]
    sc = sc[:_N]

    a = jnp.stack([1.0 + eps1, 1.0 + eps2]).reshape(1, 2)
    zz0, zz1, h2a, sums = _p1_call(
        x, c2, sx, sc, a,
        W1[:32], W1[32:], W3[:32], W3[32:],
        b1.reshape(1, 64), b3.reshape(1, 64))
    out = _p2_call(
        zz0, zz1, h2a, sums,
        W2, W4, b2.reshape(1, 64), b4.reshape(1, 64),
        g1.reshape(1, 64), be1.reshape(1, 64),
        g2.reshape(1, 64), be2.reshape(1, 64))
    return out
